# baseline (speedup 1.0000x reference)
import functools
import math

import jax
import jax.numpy as jnp
from jax import lax
from jax.experimental import pallas as pl
from jax.experimental.pallas import tpu as pltpu
from jax.experimental.pallas import tpu_sc as plsc

VOCAB = 1000
D = 128
N = 1024 * 200
LANES = 16

NUM_CORES = 2
NUM_SUBCORES = 16
NW = NUM_CORES * NUM_SUBCORES

CHUNK = 64
ROWS_PER_W = N // NW
CHUNKS_PER_W = ROWS_PER_W // CHUNK
NIN = 3
NOUT = 2
UNROLL = 6

SCALE = math.sqrt(D)


def _sc_body(table_hbm, idx_hbm, pos_hbm, out_hbm,
             table_sh, idx_v, rows_v, pos_v, out_v,
             tsem, gsem0, gsem1, gsem2, psem0, psem1, psem2, osem0, osem1):
    sid = lax.axis_index("s")
    wid = sid * NUM_CORES + lax.axis_index("c")
    base = pl.multiple_of(wid * ROWS_PER_W, CHUNK)
    gsems = [gsem0, gsem1, gsem2]
    psems = [psem0, psem1, psem2]
    osems = [osem0, osem1]

    @pl.when(sid == 0)
    def _():
        pltpu.make_async_copy(table_hbm, table_sh, tsem).start()

    pltpu.sync_copy(idx_hbm.at[pl.ds(base, ROWS_PER_W)], idx_v)

    @pl.when(sid == 0)
    def _():
        pltpu.make_async_copy(table_hbm, table_sh, tsem).wait()

    plsc.subcore_barrier()

    def in_copies(c, b):
        start = pl.multiple_of(base + c * CHUNK, CHUNK)
        idx_sl = idx_v.at[pl.ds(pl.multiple_of(c * CHUNK, CHUNK), CHUNK)]
        g = pltpu.make_async_copy(table_sh.at[idx_sl], rows_v.at[b], gsems[b])
        p = pltpu.make_async_copy(pos_hbm.at[pl.ds(start, CHUNK), :],
                                  pos_v.at[b], psems[b])
        return g, p

    def out_copy(c, b):
        start = pl.multiple_of(base + c * CHUNK, CHUNK)
        return pltpu.make_async_copy(out_v.at[b],
                                     out_hbm.at[pl.ds(start, CHUNK), :],
                                     osems[b])

    def compute(bi, bo):
        @plsc.parallel_loop(0, CHUNK, 1, unroll=4)
        def row_body(i):
            for j in range(D // LANES):
                sl = pl.ds(j * LANES, LANES)
                out_v[bo, i, sl] = (rows_v[bi, i, sl] * SCALE
                                    + pos_v[bi, i, sl])

    def body(c, k, prefetch, owait):
        bi = k % NIN
        bo = k % NOUT
        g, p = in_copies(c, bi)
        g.wait()
        p.wait()
        if owait:
            out_copy(c - NOUT, bo).wait()
        compute(bi, bo)
        out_copy(c, bo).start()
        if prefetch:
            g2, p2 = in_copies(c + NIN, bi)
            g2.start()
            p2.start()

    for b in range(NIN):
        g, p = in_copies(b, b)
        g.start()
        p.start()

    for k in range(UNROLL):
        body(k, k, prefetch=True, owait=(k >= NOUT))

    def steady(i, carry):
        for k in range(UNROLL):
            body(i * UNROLL + k, k, prefetch=True, owait=True)
        return carry

    lax.fori_loop(1, CHUNKS_PER_W // UNROLL, steady, 0)

    tail0 = (CHUNKS_PER_W // UNROLL) * UNROLL
    for c in range(tail0, CHUNKS_PER_W):
        k = c - tail0
        body(c, k, prefetch=(c + NIN < CHUNKS_PER_W), owait=True)

    for c in range(CHUNKS_PER_W - NOUT, CHUNKS_PER_W):
        out_copy(c, (c - tail0) % NOUT).wait()


@jax.jit
def _sc_embed(x_flat, position_flat, table):
    mesh = plsc.VectorSubcoreMesh(core_axis_name="c", subcore_axis_name="s")
    kern = functools.partial(
        pl.kernel,
        mesh=mesh,
        out_type=jax.ShapeDtypeStruct((N, D), jnp.float32),
        scratch_types=[
            pltpu.VMEM_SHARED((VOCAB, D), jnp.float32),
            pltpu.VMEM((ROWS_PER_W,), jnp.int32),
            pltpu.VMEM((NIN, CHUNK, D), jnp.float32),
            pltpu.VMEM((NIN, CHUNK, D), jnp.float32),
            pltpu.VMEM((NOUT, CHUNK, D), jnp.float32),
            pltpu.SemaphoreType.DMA,
            pltpu.SemaphoreType.DMA,
            pltpu.SemaphoreType.DMA,
            pltpu.SemaphoreType.DMA,
            pltpu.SemaphoreType.DMA,
            pltpu.SemaphoreType.DMA,
            pltpu.SemaphoreType.DMA,
            pltpu.SemaphoreType.DMA,
            pltpu.SemaphoreType.DMA,
        ],
    )(_sc_body)
    return kern(table, x_flat, position_flat)


def kernel(x, position, table):
    x_flat = x.reshape(N)
    pos_flat = position.reshape(N, D)
    out = _sc_embed(x_flat, pos_flat, table)
    return out.reshape(position.shape)


# speedup vs baseline: 1.0044x; 1.0044x over previous
import functools
import math

import jax
import jax.numpy as jnp
from jax import lax
from jax.experimental import pallas as pl
from jax.experimental.pallas import tpu as pltpu
from jax.experimental.pallas import tpu_sc as plsc

VOCAB = 1000
D = 128
N = 1024 * 200
LANES = 16

NUM_CORES = 2
NUM_SUBCORES = 16
NW = NUM_CORES * NUM_SUBCORES

CHUNK = 128
HALF = CHUNK // 2
ROWS_PER_W = N // NW
CHUNKS_PER_W = ROWS_PER_W // CHUNK
NIN = 3

SCALE = math.sqrt(D)


def _sc_body(table_hbm, idx_hbm, pos_hbm, out_hbm,
             table_sh, idx_v, rows_v, pos_v, out_v,
             tsem, gsem0, gsem1, gsem2, psem0, psem1, psem2, osem0, osem1):
    sid = lax.axis_index("s")
    wid = sid * NUM_CORES + lax.axis_index("c")
    base = pl.multiple_of(wid * ROWS_PER_W, CHUNK)
    gsems = [gsem0, gsem1, gsem2]
    psems = [psem0, psem1, psem2]
    osems = [osem0, osem1]

    @pl.when(sid == 0)
    def _():
        pltpu.make_async_copy(table_hbm, table_sh, tsem).start()

    pltpu.sync_copy(idx_hbm.at[pl.ds(base, ROWS_PER_W)], idx_v)

    @pl.when(sid == 0)
    def _():
        pltpu.make_async_copy(table_hbm, table_sh, tsem).wait()

    plsc.subcore_barrier()

    def in_copies(c, b):
        start = pl.multiple_of(base + c * CHUNK, CHUNK)
        idx_sl = idx_v.at[pl.ds(pl.multiple_of(c * CHUNK, CHUNK), CHUNK)]
        g = pltpu.make_async_copy(table_sh.at[idx_sl], rows_v.at[b], gsems[b])
        p = pltpu.make_async_copy(pos_hbm.at[pl.ds(start, CHUNK), :],
                                  pos_v.at[b], psems[b])
        return g, p

    def out_copy(c, h):
        start = pl.multiple_of(base + c * CHUNK + h * HALF, HALF)
        return pltpu.make_async_copy(out_v.at[h],
                                     out_hbm.at[pl.ds(start, HALF), :],
                                     osems[h])

    def compute_half(bi, h):
        @plsc.parallel_loop(0, HALF, 1, unroll=4)
        def row_body(i):
            r = i + h * HALF
            for j in range(D // LANES):
                sl = pl.ds(j * LANES, LANES)
                out_v[h, i, sl] = (rows_v[bi, r, sl] * SCALE
                                   + pos_v[bi, r, sl])

    def body(c, bi, prefetch, owait):
        g, p = in_copies(c, bi)
        g.wait()
        p.wait()
        for h in range(2):
            if owait:
                out_copy(c - 1, h).wait()
            compute_half(bi, h)
            out_copy(c, h).start()
        if prefetch:
            g2, p2 = in_copies(c + NIN, bi)
            g2.start()
            p2.start()

    for b in range(NIN):
        g, p = in_copies(b, b)
        g.start()
        p.start()

    for k in range(NIN):
        body(k, k, prefetch=True, owait=(k > 0))

    def steady(i, carry):
        for k in range(NIN):
            body(i * NIN + k, k, prefetch=True, owait=True)
        return carry

    lax.fori_loop(1, CHUNKS_PER_W // NIN - 1, steady, 0)

    for c in range(CHUNKS_PER_W - 5, CHUNKS_PER_W):
        body(c, c % NIN, prefetch=(c + NIN < CHUNKS_PER_W), owait=True)

    for h in range(2):
        out_copy(CHUNKS_PER_W - 1, h).wait()


@jax.jit
def _sc_embed(x_flat, position_flat, table):
    mesh = plsc.VectorSubcoreMesh(core_axis_name="c", subcore_axis_name="s")
    kern = functools.partial(
        pl.kernel,
        mesh=mesh,
        out_type=jax.ShapeDtypeStruct((N, D), jnp.float32),
        scratch_types=[
            pltpu.VMEM_SHARED((VOCAB, D), jnp.float32),
            pltpu.VMEM((ROWS_PER_W,), jnp.int32),
            pltpu.VMEM((NIN, CHUNK, D), jnp.float32),
            pltpu.VMEM((NIN, CHUNK, D), jnp.float32),
            pltpu.VMEM((2, HALF, D), jnp.float32),
            pltpu.SemaphoreType.DMA,
            pltpu.SemaphoreType.DMA,
            pltpu.SemaphoreType.DMA,
            pltpu.SemaphoreType.DMA,
            pltpu.SemaphoreType.DMA,
            pltpu.SemaphoreType.DMA,
            pltpu.SemaphoreType.DMA,
            pltpu.SemaphoreType.DMA,
            pltpu.SemaphoreType.DMA,
        ],
    )(_sc_body)
    return kern(table, x_flat, position_flat)


def kernel(x, position, table):
    x_flat = x.reshape(N)
    pos_flat = position.reshape(N, D)
    out = _sc_embed(x_flat, pos_flat, table)
    return out.reshape(position.shape)
